# baseline (speedup 1.0000x reference)
import functools

import jax
import jax.numpy as jnp
from jax import lax
from jax.experimental import pallas as pl
from jax.experimental.pallas import tpu as pltpu
from jax.experimental.pallas import tpu_sc as plsc

_B = 4
_S = 2048
_NSTRIPE = 8
_SR = _S // _NSTRIPE
_R = 16
_NCHUNK = _SR // _R
_CB = _SR


def _sc_body(x_hbm, out_hbm, buf, sem):
    cid = lax.axis_index("c")
    sid = lax.axis_index("s")
    wid = sid * 2 + cid
    b = wid // _NSTRIPE
    st = wid % _NSTRIPE
    row0 = st * _SR

    ones16 = jnp.full((16,), 1.0, jnp.float32)

    def _fill_row(r, _):
        def _fill_g(g, _):
            buf[r, pl.ds(g * 16, 16)] = ones16
            return 0

        return lax.fori_loop(0, _S // 16, _fill_g, 0)

    lax.fori_loop(0, _R, _fill_row, 0)

    def _chunk(c, _):
        r0 = row0 + c * _R

        def _dma_j(j, _):
            pltpu.async_copy(
                x_hbm.at[b, pl.ds(r0, _R), pl.ds(j * _CB, _CB)],
                buf.at[:, pl.ds(j * _CB, _CB)],
                sem,
            ).wait()
            return 0

        lax.fori_loop(0, st, _dma_j, 0)
        pltpu.async_copy(
            x_hbm.at[b, pl.ds(r0, _R), pl.ds(st * _CB, _CB)],
            buf.at[:, pl.ds(st * _CB, _CB)],
            sem,
        ).wait()

        for rl in range(_R):
            rg = r0 + rl
            for g in range(_CB // 16):
                c0 = st * _CB + g * 16
                colv = c0 + lax.iota(jnp.int32, 16)
                v = buf[rl, pl.ds(c0, 16)]
                buf[rl, pl.ds(c0, 16)] = jnp.where(colv > rg, 1.0, v)

        pltpu.async_copy(
            buf,
            out_hbm.at[b, pl.ds(r0, _R), :],
            sem,
        ).wait()
        return 0

    lax.fori_loop(0, _NCHUNK, _chunk, 0)


def kernel(x):
    mesh = plsc.VectorSubcoreMesh(core_axis_name="c", subcore_axis_name="s")
    fn = functools.partial(
        pl.kernel,
        mesh=mesh,
        out_type=jax.ShapeDtypeStruct((_B, _S, _S), jnp.float32),
        scratch_types=[
            pltpu.VMEM((_R, _S), jnp.float32),
            pltpu.SemaphoreType.DMA,
        ],
    )(_sc_body)
    return fn(x)


# speedup vs baseline: 4.5405x; 4.5405x over previous
import jax
import jax.numpy as jnp
from jax.experimental import pallas as pl
from jax.experimental.pallas import tpu as pltpu

_RB = 512
_N = 4
_S = 2048


def _mask_kernel(*refs):
    xs = refs[:_N]
    o_ref = refs[_N]
    i = pl.program_id(1)
    for k in range(_N):
        w = (k + 1) * _RB

        @pl.when(i == k)
        def _stripe(k=k, x_ref=xs[k], w=w):
            rows = k * _RB + jax.lax.broadcasted_iota(jnp.int32, (1, _RB, w), 1)
            cols = jax.lax.broadcasted_iota(jnp.int32, (1, _RB, w), 2)
            o_ref[:, :, :w] = jnp.where(cols > rows, jnp.float32(1.0), x_ref[...])
            if w < _S:
                o_ref[:, :, w:] = jnp.ones((1, _RB, _S - w), o_ref.dtype)


def kernel(x):
    B, S, _ = x.shape
    grid = (B, _N)

    def _in_map(k):
        def _map(b, i, k=k):
            b_eff = jnp.minimum(b + (i > k).astype(b.dtype), B - 1)
            return (b_eff, k, 0)

        return _map

    in_specs = [
        pl.BlockSpec((1, _RB, (k + 1) * _RB), _in_map(k)) for k in range(_N)
    ]
    return pl.pallas_call(
        _mask_kernel,
        grid=grid,
        in_specs=in_specs,
        out_specs=pl.BlockSpec((1, _RB, S), lambda b, i: (b, i, 0)),
        out_shape=jax.ShapeDtypeStruct(x.shape, x.dtype),
        compiler_params=pltpu.CompilerParams(
            dimension_semantics=("arbitrary", "arbitrary"),
        ),
    )(*([x] * _N))
